# Initial kernel scaffold; baseline (speedup 1.0000x reference)
import jax
import jax.numpy as jnp
from jax.experimental import pallas as pl


def kernel(x8, x4, W1, b1, prelu_a, W2, b2):
    raise NotImplementedError("write your pallas kernel here")


# speedup vs baseline: 13.4826x; 13.4826x over previous
import functools

import jax
import jax.numpy as jnp
from jax import lax
from jax.experimental import pallas as pl
from jax.experimental.pallas import tpu as pltpu
from jax.experimental.pallas import tpu_sc as plsc

C = 96
H = W = 224
LH = 75
L = LH * LH
LP = 5632
CK = 864
CKP = 896

QT = 256
NQT = LP // QT

NPIX = H * W
PT = 6272

WPAD = 232
HHP = 226
C2T = 6560
C2N = 8
C2FLAT = C2T * (C2N + 1)
OFFS = tuple(WPAD * dy + dx for dy in range(3) for dx in range(3))

_BIG_I = 1 << 30


def _knorm_body(k_ref, o_ref):
    x = k_ref[...]
    n = jnp.sqrt(jnp.sum(x * x, axis=0, keepdims=True))
    o_ref[...] = x / jnp.maximum(n, 1e-12)


def _normalize_keys(kraw):
    return pl.pallas_call(
        _knorm_body,
        grid=(11,),
        in_specs=[pl.BlockSpec((CKP, LP // 11), lambda i: (0, i))],
        out_specs=pl.BlockSpec((CKP, LP // 11), lambda i: (0, i)),
        out_shape=jax.ShapeDtypeStruct((CKP, LP), jnp.float32),
    )(kraw)


def _corr_body(q_ref, kn_ref, id2_ref):
    q = q_ref[...]
    rn = jnp.sqrt(jnp.sum(q * q, axis=1, keepdims=True))
    qn = q / jnp.maximum(rn, 1e-12)
    s = jax.lax.dot_general(
        qn, kn_ref[...], (((1,), (0,)), ((), ())),
        preferred_element_type=jnp.float32)
    col = lax.broadcasted_iota(jnp.int32, s.shape, 1)
    s = jnp.where(col < L, s, -2.0)
    m1 = jnp.max(s, axis=1, keepdims=True)
    i1 = jnp.min(jnp.where(s == m1, col, _BIG_I), axis=1, keepdims=True)
    s2 = jnp.where(col == i1, -3.0, s)
    m2 = jnp.max(s2, axis=1, keepdims=True)
    i2 = jnp.min(jnp.where(s2 == m2, col, _BIG_I), axis=1, keepdims=True)
    id2_ref[0, 0, :] = i2[:, 0]


def _top2_indices(qraw, kn):
    id2 = pl.pallas_call(
        _corr_body,
        grid=(NQT,),
        in_specs=[
            pl.BlockSpec((QT, CKP), lambda i: (i, 0)),
            pl.BlockSpec((CKP, LP), lambda i: (0, 0)),
        ],
        out_specs=pl.BlockSpec((1, 1, QT), lambda i: (i, 0, 0)),
        out_shape=jax.ShapeDtypeStruct((NQT, 1, QT), jnp.int32),
    )(qraw, kn)
    return id2.reshape(LP)


def _sc_gather(table, idx):
    try:
        info = plsc.get_sparse_core_info()
        nc, ns = info.num_cores, info.num_subcores
    except Exception:
        nc, ns = 2, 16
    nw = nc * ns
    bpw = LP // nw
    nch = 2
    gr = bpw // nch

    @functools.partial(
        pl.kernel,
        out_type=jax.ShapeDtypeStruct((LP, CKP), jnp.float32),
        mesh=plsc.VectorSubcoreMesh(core_axis_name="c", subcore_axis_name="s"),
        scratch_types=[
            pltpu.VMEM((gr,), jnp.int32),
            pltpu.VMEM((gr, CKP), jnp.float32),
            pltpu.SemaphoreType.DMA,
        ],
    )
    def gather_kernel(table_hbm, idx_hbm, out_hbm, idx_v, rows_v, sem):
        wid = lax.axis_index("s") * nc + lax.axis_index("c")
        base = wid * bpw
        for ch in range(nch):
            b = base + ch * gr
            pltpu.sync_copy(idx_hbm.at[pl.ds(b, gr)], idx_v)
            pltpu.async_copy(table_hbm.at[idx_v], rows_v, sem).wait()
            pltpu.sync_copy(rows_v, out_hbm.at[pl.ds(b, gr)])

    return gather_kernel(table, idx)


def _c1_body(x_ref, t_ref, wa_ref, wb_ref, b_ref, a_ref, o_ref):
    y = jax.lax.dot_general(
        x_ref[...], wa_ref[...], (((1,), (0,)), ((), ())),
        preferred_element_type=jnp.float32)
    y = y + jax.lax.dot_general(
        t_ref[...], wb_ref[...], (((1,), (0,)), ((), ())),
        preferred_element_type=jnp.float32)
    y = y + b_ref[...]
    a = a_ref[0, 0]
    o_ref[...] = jnp.where(y >= 0, y, a * y)


def _conv1(x_pix, t_pix, wa, wb, b1, a):
    return pl.pallas_call(
        _c1_body,
        grid=(NPIX // PT,),
        in_specs=[
            pl.BlockSpec((PT, C), lambda i: (i, 0)),
            pl.BlockSpec((PT, C), lambda i: (i, 0)),
            pl.BlockSpec((C, C), lambda i: (0, 0)),
            pl.BlockSpec((C, C), lambda i: (0, 0)),
            pl.BlockSpec((1, C), lambda i: (0, 0)),
            pl.BlockSpec((1, 1), lambda i: (0, 0)),
        ],
        out_specs=pl.BlockSpec((PT, C), lambda i: (i, 0)),
        out_shape=jax.ShapeDtypeStruct((NPIX, C), jnp.float32),
    )(x_pix, t_pix, wa, wb, b1, a)


def _c2_body(ha_ref, hb_ref, w_ref, b_ref, y_ref):
    acc = jnp.zeros((C2T, C), jnp.float32)
    for si, off in enumerate(OFFS):
        if off == 0:
            hs = ha_ref[...]
        else:
            hs = jnp.concatenate([ha_ref[off:, :], hb_ref[:off, :]], axis=0)
        acc = acc + jax.lax.dot_general(
            hs, w_ref[si], (((1,), (0,)), ((), ())),
            preferred_element_type=jnp.float32)
    y_ref[...] = acc + b_ref[...]


def _conv2(hflat, w2s, b2):
    return pl.pallas_call(
        _c2_body,
        grid=(C2N,),
        in_specs=[
            pl.BlockSpec((C2T, C), lambda i: (i, 0)),
            pl.BlockSpec((C2T, C), lambda i: (i + 1, 0)),
            pl.BlockSpec((9, C, C), lambda i: (0, 0, 0)),
            pl.BlockSpec((1, C), lambda i: (0, 0)),
        ],
        out_specs=pl.BlockSpec((C2T, C), lambda i: (i, 0)),
        out_shape=jax.ShapeDtypeStruct((C2N * C2T, C), jnp.float32),
    )(hflat, hflat, w2s, b2)


def kernel(x8, x4, W1, b1, prelu_a, W2, b2):
    f32 = jnp.float32
    x4i = x4[0].astype(f32)
    x8i = x8[0].astype(f32)

    x4p = jnp.pad(x4i, ((0, 0), (1, 1), (1, 1)))[:, :225, :225]
    x8p = jnp.pad(x8i, ((0, 0), (1, 1), (1, 1)))[:, :225, :225]

    k5 = x4p.reshape(C, LH, 3, LH, 3)
    kraw = k5.transpose(0, 2, 4, 1, 3).reshape(CK, L)
    kraw = jnp.pad(kraw, ((0, CKP - CK), (0, LP - L)))

    q5 = x8p.reshape(C, LH, 3, LH, 3)
    qraw = q5.transpose(1, 3, 0, 2, 4).reshape(L, CK)
    qraw = jnp.pad(qraw, ((0, LP - L), (0, CKP - CK)))

    kn = _normalize_keys(kraw)
    id2 = _top2_indices(qraw, kn)

    xt = x4p.transpose(1, 2, 0)
    table = xt.reshape(LH, 3, LH, 3, C).transpose(0, 2, 1, 3, 4).reshape(L, CK)
    table = jnp.pad(table, ((0, LP - L), (0, CKP - CK)))

    tr = _sc_gather(table, id2)

    t225 = tr[:L, :CK].reshape(LH, LH, 3, 3, C).transpose(0, 2, 1, 3, 4)
    t225 = t225.reshape(225, 225, C)
    t_pix = t225[1:225, 1:225, :].reshape(NPIX, C)
    x4_pix = x4i.transpose(1, 2, 0).reshape(NPIX, C)

    w1t = W1[:, :, 0, 0].T.astype(f32)
    h_pix = _conv1(x4_pix, t_pix, w1t[:C], w1t[C:], b1.reshape(1, C),
                   prelu_a.reshape(1, 1))

    h_img = h_pix.reshape(H, W, C)
    hp = jnp.pad(h_img, ((1, 1), (1, WPAD - W - 1), (0, 0)))
    hflat = hp.reshape(HHP * WPAD, C)
    hflat = jnp.pad(hflat, ((0, C2FLAT - HHP * WPAD), (0, 0)))

    w2s = W2.transpose(2, 3, 1, 0).reshape(9, C, C)
    y = _conv2(hflat, w2s, b2.reshape(1, C))

    out = y[: HHP * WPAD].reshape(HHP, WPAD, C)[:H, :W, :]
    return out.transpose(2, 0, 1)[None].astype(x8.dtype)
